# Initial kernel scaffold; baseline (speedup 1.0000x reference)
import jax
import jax.numpy as jnp
from jax.experimental import pallas as pl


def kernel(y, depths, adversaries):
    raise NotImplementedError("write your pallas kernel here")


# speedup vs baseline: 1.9702x; 1.9702x over previous
import functools

import jax
import jax.numpy as jnp
from jax import lax
from jax.experimental import pallas as pl
from jax.experimental.pallas import tpu as pltpu
from jax.experimental.pallas import tpu_sc as plsc

N_LABELS = 4096
MAX_DEPTH = 3
BATCH = 16384

NC = 2
NS = 16
L = 16
NW = NC * NS
BPW = BATCH // NW
R = 16
NCH = BPW // R

_mesh = plsc.VectorSubcoreMesh(core_axis_name="c", subcore_axis_name="s")


@functools.partial(
    pl.kernel,
    mesh=_mesh,
    out_type=jax.ShapeDtypeStruct((BATCH, N_LABELS), jnp.bool_),
    scratch_types=[
        pltpu.VMEM((BPW,), jnp.int32),
        pltpu.VMEM((BPW,), jnp.int32),
        pltpu.VMEM((BPW,), jnp.int32),
        pltpu.VMEM((R, N_LABELS), jnp.int32),
        pltpu.SemaphoreType.DMA,
    ],
)
def _gather_rows(yl_hbm, d_hbm, table_hbm, out_hbm, idx_v, yl_v, d_v, buf, sem):
    wid = lax.axis_index("s") * NC + lax.axis_index("c")
    base = wid * BPW
    pltpu.sync_copy(yl_hbm.at[pl.ds(base, BPW)], yl_v)
    pltpu.sync_copy(d_hbm.at[pl.ds(base, BPW)], d_v)

    def idx_body(i, carry):
        s = pl.ds(i * L, L)
        idx_v[s] = d_v[s] * N_LABELS + yl_v[s]
        return carry

    lax.fori_loop(0, BPW // L, idx_body, 0)

    for c in range(NCH):
        pltpu.async_copy(table_hbm.at[idx_v.at[pl.ds(c * R, R)]], buf, sem).wait()
        pltpu.sync_copy(buf, out_hbm.at[pl.ds(base + c * R, R)])


def kernel(y, depths, adversaries):
    table = adversaries.reshape(MAX_DEPTH * N_LABELS, N_LABELS)
    y_leaf = y[:, MAX_DEPTH - 1]
    d = depths[:, 0]
    return _gather_rows(y_leaf, d, table)
